# baseline (speedup 1.0000x reference)
import functools

import jax
import jax.numpy as jnp
from jax import lax
from jax.experimental import pallas as pl
from jax.experimental.pallas import tpu as pltpu
from jax.experimental.pallas import tpu_sc as plsc

E = 8
TOP_K = 2
H = 2048
F = 1024
T = 2048
TM = 128
NMAX = T * TOP_K + E * TM
NT = NMAX // TM

_NC = 2
_NS = 16
_NW = _NC * _NS
_TOK_W = T // _NW
_CH = 16
_NCH = _TOK_W // _CH


def _router_body(x_ref, gwT_ref, logits_ref, w0_ref, w1_ref,
                 pos0_ref, pos1_ref, sizes_ref, xbf_ref):
    x = x_ref[...]
    xb = x.astype(jnp.bfloat16)
    xbf_ref[...] = xb
    logits = jnp.dot(xb, gwT_ref[...].astype(jnp.bfloat16),
                     preferred_element_type=jnp.float32)
    logits_ref[...] = logits
    p = jax.nn.softmax(logits, axis=-1)
    lane = lax.broadcasted_iota(jnp.int32, p.shape, 1)
    p1 = jnp.max(p, axis=-1, keepdims=True)
    i1 = jnp.min(jnp.where(p == p1, lane, E), axis=-1, keepdims=True)
    m1 = lane == i1
    p_no1 = jnp.where(m1, -jnp.inf, p)
    p2 = jnp.max(p_no1, axis=-1, keepdims=True)
    i2 = jnp.min(jnp.where(p_no1 == p2, lane, E), axis=-1, keepdims=True)
    m2 = lane == i2
    denom = p1 + p2
    w0_ref[...] = p1 / denom
    w1_ref[...] = p2 / denom

    C = jnp.where(m1 | m2, 1.0, 0.0)
    r128 = lax.broadcasted_iota(jnp.int32, (TM, TM), 0)
    c128 = lax.broadcasted_iota(jnp.int32, (TM, TM), 1)
    L128 = jnp.where(r128 > c128, 1.0, 0.0).astype(jnp.bfloat16)
    excl_rows = []
    acc = jnp.zeros((1, E), jnp.float32)
    for b in range(T // TM):
        C_sl = C[b * TM:(b + 1) * TM]
        intra = jnp.dot(L128, C_sl.astype(jnp.bfloat16),
                        preferred_element_type=jnp.float32)
        excl_rows.append(intra + acc)
        acc = acc + jnp.sum(C_sl, axis=0, keepdims=True)
    excl = jnp.concatenate(excl_rows, axis=0)
    sizes = acc

    q = jnp.ceil(sizes / TM)
    r8 = lax.broadcasted_iota(jnp.int32, (E, E), 0)
    c8 = lax.broadcasted_iota(jnp.int32, (E, E), 1)
    U8 = jnp.where(r8 < c8, 1.0, 0.0).astype(jnp.bfloat16)
    starts = TM * jnp.dot(q.astype(jnp.bfloat16), U8,
                          preferred_element_type=jnp.float32)

    starts_b = jnp.broadcast_to(starts, (T, E))
    sel1 = jnp.sum(jnp.where(lane == i1, starts_b + excl, 0.0),
                   axis=-1, keepdims=True)
    sel2 = jnp.sum(jnp.where(lane == i2, starts_b + excl, 0.0),
                   axis=-1, keepdims=True)
    pos0_ref[...] = sel1.astype(jnp.int32)
    pos1_ref[...] = sel2.astype(jnp.int32)
    sizes_ref[...] = sizes.astype(jnp.int32)


def _run_router(hs, gwT):
    return pl.pallas_call(
        _router_body,
        in_specs=[
            pl.BlockSpec((T, H), lambda: (0, 0)),
            pl.BlockSpec((H, E), lambda: (0, 0)),
        ],
        out_specs=[
            pl.BlockSpec((T, E), lambda: (0, 0)),
            pl.BlockSpec((T, 1), lambda: (0, 0)),
            pl.BlockSpec((T, 1), lambda: (0, 0)),
            pl.BlockSpec((T, 1), lambda: (0, 0)),
            pl.BlockSpec((T, 1), lambda: (0, 0)),
            pl.BlockSpec((1, E), lambda: (0, 0)),
            pl.BlockSpec((T, H), lambda: (0, 0)),
        ],
        out_shape=[
            jax.ShapeDtypeStruct((T, E), jnp.float32),
            jax.ShapeDtypeStruct((T, 1), jnp.float32),
            jax.ShapeDtypeStruct((T, 1), jnp.float32),
            jax.ShapeDtypeStruct((T, 1), jnp.int32),
            jax.ShapeDtypeStruct((T, 1), jnp.int32),
            jax.ShapeDtypeStruct((1, E), jnp.int32),
            jax.ShapeDtypeStruct((T, H), jnp.bfloat16),
        ],
    )(hs, gwT)


@functools.cache
def _make_sc_kernels():
    mesh = plsc.VectorSubcoreMesh(core_axis_name="c", subcore_axis_name="s")

    @functools.partial(
        pl.kernel, mesh=mesh,
        out_type=jax.ShapeDtypeStruct((NMAX, H), jnp.float32),
        scratch_types=[
            pltpu.VMEM((_CH, H), jnp.float32),
            pltpu.VMEM((2 * _NCH, _CH), jnp.int32),
            pltpu.SemaphoreType.DMA,
            pltpu.SemaphoreType.DMA,
        ],
    )
    def dispatch(hs_hbm, pos0_hbm, pos1_hbm, xs_hbm, rows_v, idx_v, sem0, sem1):
        wid = lax.axis_index("s") * _NC + lax.axis_index("c")
        base = wid * _TOK_W
        for j in range(_NCH):
            off = base + j * _CH
            pltpu.sync_copy(pos0_hbm.at[pl.ds(off, _CH)], idx_v.at[j])
            pltpu.sync_copy(pos1_hbm.at[pl.ds(off, _CH)], idx_v.at[_NCH + j])
            pltpu.sync_copy(hs_hbm.at[pl.ds(off, _CH)], rows_v)
            cp0 = pltpu.async_copy(rows_v, xs_hbm.at[idx_v.at[j]], sem0)
            cp1 = pltpu.async_copy(rows_v, xs_hbm.at[idx_v.at[_NCH + j]], sem1)
            cp0.wait()
            cp1.wait()

    @functools.partial(
        pl.kernel, mesh=mesh,
        out_type=[
            jax.ShapeDtypeStruct((T, H), jnp.float32),
            jax.ShapeDtypeStruct((T, H), jnp.float32),
        ],
        scratch_types=[
            pltpu.VMEM((_CH, H), jnp.float32),
            pltpu.VMEM((_CH, H), jnp.float32),
            pltpu.VMEM((2 * _NCH, _CH), jnp.int32),
            pltpu.SemaphoreType.DMA,
            pltpu.SemaphoreType.DMA,
        ],
    )
    def combine(y_hbm, pos0_hbm, pos1_hbm, y0_hbm, y1_hbm,
                rows0_v, rows1_v, idx_v, sem0, sem1):
        wid = lax.axis_index("s") * _NC + lax.axis_index("c")
        base = wid * _TOK_W
        for j in range(_NCH):
            off = base + j * _CH
            pltpu.sync_copy(pos0_hbm.at[pl.ds(off, _CH)], idx_v.at[j])
            pltpu.sync_copy(pos1_hbm.at[pl.ds(off, _CH)], idx_v.at[_NCH + j])
            cp0 = pltpu.async_copy(y_hbm.at[idx_v.at[j]], rows0_v, sem0)
            cp1 = pltpu.async_copy(y_hbm.at[idx_v.at[_NCH + j]], rows1_v, sem1)
            cp0.wait()
            cp1.wait()
            pltpu.sync_copy(rows0_v, y0_hbm.at[pl.ds(off, _CH)])
            pltpu.sync_copy(rows1_v, y1_hbm.at[pl.ds(off, _CH)])

    return dispatch, combine


def _dispatch_sc(hs, p0, p1):
    return _make_sc_kernels()[0](hs, p0, p1)


def _combine_sc(y, p0, p1):
    return _make_sc_kernels()[1](y, p0, p1)


BFA = 128
NFA = F // BFA
BHB = 256
NHB = H // BHB


def _gemm_a_body(st_ref, x_ref, wg_ref, wu_ref, h_ref):
    e = pl.program_id(1)
    row0 = st_ref[e]
    ntiles = (st_ref[e + 1] - row0) // TM
    wg = wg_ref[0].astype(jnp.bfloat16)
    wu = wu_ref[0].astype(jnp.bfloat16)

    def tile(i, _):
        r = pl.multiple_of(row0 + i * TM, TM)
        xb = x_ref[pl.ds(r, TM), :].astype(jnp.bfloat16)
        g = jnp.dot(xb, wg, preferred_element_type=jnp.float32)
        u = jnp.dot(xb, wu, preferred_element_type=jnp.float32)
        h_ref[pl.ds(r, TM), :] = (u * (g * jax.nn.sigmoid(g))).astype(jnp.bfloat16)
        return 0

    lax.fori_loop(0, ntiles, tile, 0)


def _gemm_b_body(st_ref, h_ref, wd_ref, y_ref):
    e = pl.program_id(1)
    row0 = st_ref[e]
    ntiles = (st_ref[e + 1] - row0) // TM
    wd = wd_ref[0].astype(jnp.bfloat16)

    def tile(i, _):
        r = pl.multiple_of(row0 + i * TM, TM)
        y_ref[pl.ds(r, TM), :] = jnp.dot(h_ref[pl.ds(r, TM), :], wd,
                                         preferred_element_type=jnp.float32)
        return 0

    lax.fori_loop(0, ntiles, tile, 0)


def _run_gemms(starts, x_sorted, gate_up_proj, down_proj):
    h = pl.pallas_call(
        _gemm_a_body,
        grid_spec=pltpu.PrefetchScalarGridSpec(
            num_scalar_prefetch=1,
            grid=(NFA, E),
            in_specs=[
                pl.BlockSpec((NMAX, H), lambda f, e, st: (0, 0)),
                pl.BlockSpec((1, H, BFA), lambda f, e, st: (e, 0, f)),
                pl.BlockSpec((1, H, BFA), lambda f, e, st: (e, 0, f + NFA)),
            ],
            out_specs=pl.BlockSpec((NMAX, BFA), lambda f, e, st: (0, f)),
        ),
        out_shape=jax.ShapeDtypeStruct((NMAX, F), jnp.bfloat16),
    )(starts, x_sorted, gate_up_proj, gate_up_proj)

    y = pl.pallas_call(
        _gemm_b_body,
        grid_spec=pltpu.PrefetchScalarGridSpec(
            num_scalar_prefetch=1,
            grid=(NHB, E),
            in_specs=[
                pl.BlockSpec((NMAX, F), lambda hb, e, st: (0, 0)),
                pl.BlockSpec((1, F, BHB), lambda hb, e, st: (e, 0, hb)),
            ],
            out_specs=pl.BlockSpec((NMAX, BHB), lambda hb, e, st: (0, hb)),
        ),
        out_shape=jax.ShapeDtypeStruct((NMAX, H), jnp.float32),
    )(starts, h, down_proj)
    return y


def _finish_body(w0_ref, w1_ref, y0_ref, y1_ref, out_ref):
    out_ref[...] = w0_ref[...] * y0_ref[...] + w1_ref[...] * y1_ref[...]


def _run_finish(w0, w1, y0, y1):
    bt = T // 4
    return pl.pallas_call(
        _finish_body,
        grid=(4,),
        in_specs=[
            pl.BlockSpec((bt, 1), lambda i: (i, 0)),
            pl.BlockSpec((bt, 1), lambda i: (i, 0)),
            pl.BlockSpec((bt, H), lambda i: (i, 0)),
            pl.BlockSpec((bt, H), lambda i: (i, 0)),
        ],
        out_specs=pl.BlockSpec((bt, H), lambda i: (i, 0)),
        out_shape=jax.ShapeDtypeStruct((T, H), jnp.float32),
    )(w0, w1, y0, y1)


def kernel(hidden_states, gate_weight, gate_up_proj, down_proj):
    B, S, _ = hidden_states.shape
    hs = hidden_states.reshape(T, H)
    gwT = gate_weight.T

    logits, w0, w1, pos0, pos1, sizes, xbf = _run_router(hs, gwT)

    sizes_v = sizes[0]
    psz = ((sizes_v + TM - 1) // TM) * TM
    ends = jnp.cumsum(psz)
    starts = jnp.concatenate(
        [jnp.zeros((1,), jnp.int32), ends.astype(jnp.int32)])

    p0 = pos0.reshape(T)
    p1 = pos1.reshape(T)

    x_sorted = _dispatch_sc(hs, p0, p1)
    y = _run_gemms(starts, x_sorted, gate_up_proj, down_proj)
    y0, y1 = _combine_sc(y, p0, p1)
    out = _run_finish(w0, w1, y0, y1)

    return out.reshape(B, S, H), logits


# speedup vs baseline: 1.3401x; 1.3401x over previous
import functools

import jax
import jax.numpy as jnp
from jax import lax
from jax.experimental import pallas as pl
from jax.experimental.pallas import tpu as pltpu
from jax.experimental.pallas import tpu_sc as plsc

E = 8
TOP_K = 2
H = 2048
F = 1024
T = 2048
TM = 128
NMAX = T * TOP_K + E * TM
NT = NMAX // TM

_NC = 2
_NS = 16
_NW = _NC * _NS
_TOK_W = T // _NW
_CH = 16
_NCH = _TOK_W // _CH


def _router_body(x_ref, gwT_ref, logits_ref, w0_ref, w1_ref,
                 pos0_ref, pos1_ref, sizes_ref, xbf_ref):
    x = x_ref[...]
    xb = x.astype(jnp.bfloat16)
    xbf_ref[...] = xb
    logits = jnp.dot(xb, gwT_ref[...].astype(jnp.bfloat16),
                     preferred_element_type=jnp.float32)
    logits_ref[...] = logits
    p = jax.nn.softmax(logits, axis=-1)
    lane = lax.broadcasted_iota(jnp.int32, p.shape, 1)
    p1 = jnp.max(p, axis=-1, keepdims=True)
    i1 = jnp.min(jnp.where(p == p1, lane, E), axis=-1, keepdims=True)
    m1 = lane == i1
    p_no1 = jnp.where(m1, -jnp.inf, p)
    p2 = jnp.max(p_no1, axis=-1, keepdims=True)
    i2 = jnp.min(jnp.where(p_no1 == p2, lane, E), axis=-1, keepdims=True)
    m2 = lane == i2
    denom = p1 + p2
    w0_ref[...] = jnp.broadcast_to(p1 / denom, (T, 16))
    w1_ref[...] = jnp.broadcast_to(p2 / denom, (T, 16))

    C = jnp.where(m1 | m2, 1.0, 0.0)
    r128 = lax.broadcasted_iota(jnp.int32, (TM, TM), 0)
    c128 = lax.broadcasted_iota(jnp.int32, (TM, TM), 1)
    L128 = jnp.where(r128 > c128, 1.0, 0.0).astype(jnp.bfloat16)
    excl_rows = []
    acc = jnp.zeros((1, E), jnp.float32)
    for b in range(T // TM):
        C_sl = C[b * TM:(b + 1) * TM]
        intra = jnp.dot(L128, C_sl.astype(jnp.bfloat16),
                        preferred_element_type=jnp.float32)
        excl_rows.append(intra + acc)
        acc = acc + jnp.sum(C_sl, axis=0, keepdims=True)
    excl = jnp.concatenate(excl_rows, axis=0)
    sizes = acc

    q = jnp.ceil(sizes / TM)
    r8 = lax.broadcasted_iota(jnp.int32, (E, E), 0)
    c8 = lax.broadcasted_iota(jnp.int32, (E, E), 1)
    U8 = jnp.where(r8 < c8, 1.0, 0.0).astype(jnp.bfloat16)
    starts = TM * jnp.dot(q.astype(jnp.bfloat16), U8,
                          preferred_element_type=jnp.float32)

    starts_b = jnp.broadcast_to(starts, (T, E))
    sel1 = jnp.sum(jnp.where(lane == i1, starts_b + excl, 0.0),
                   axis=-1, keepdims=True)
    sel2 = jnp.sum(jnp.where(lane == i2, starts_b + excl, 0.0),
                   axis=-1, keepdims=True)
    pos0_ref[...] = sel1.astype(jnp.int32)
    pos1_ref[...] = sel2.astype(jnp.int32)
    sizes_ref[...] = sizes.astype(jnp.int32)


def _run_router(hs, gwT):
    return pl.pallas_call(
        _router_body,
        in_specs=[
            pl.BlockSpec((T, H), lambda: (0, 0)),
            pl.BlockSpec((H, E), lambda: (0, 0)),
        ],
        out_specs=[
            pl.BlockSpec((T, E), lambda: (0, 0)),
            pl.BlockSpec((T, 16), lambda: (0, 0)),
            pl.BlockSpec((T, 16), lambda: (0, 0)),
            pl.BlockSpec((T, 1), lambda: (0, 0)),
            pl.BlockSpec((T, 1), lambda: (0, 0)),
            pl.BlockSpec((1, E), lambda: (0, 0)),
            pl.BlockSpec((T, H), lambda: (0, 0)),
        ],
        out_shape=[
            jax.ShapeDtypeStruct((T, E), jnp.float32),
            jax.ShapeDtypeStruct((T, 16), jnp.float32),
            jax.ShapeDtypeStruct((T, 16), jnp.float32),
            jax.ShapeDtypeStruct((T, 1), jnp.int32),
            jax.ShapeDtypeStruct((T, 1), jnp.int32),
            jax.ShapeDtypeStruct((1, E), jnp.int32),
            jax.ShapeDtypeStruct((T, H), jnp.bfloat16),
        ],
    )(hs, gwT)


@functools.cache
def _make_sc_kernels():
    mesh = plsc.VectorSubcoreMesh(core_axis_name="c", subcore_axis_name="s")

    @functools.partial(
        pl.kernel, mesh=mesh,
        out_type=jax.ShapeDtypeStruct((NMAX, H), jnp.float32),
        scratch_types=[
            pltpu.VMEM((_CH, H), jnp.float32),
            pltpu.VMEM((2 * _NCH, _CH), jnp.int32),
            pltpu.SemaphoreType.DMA,
            pltpu.SemaphoreType.DMA,
        ],
    )
    def dispatch(hs_hbm, pos0_hbm, pos1_hbm, xs_hbm, rows_v, idx_v, sem0, sem1):
        wid = lax.axis_index("s") * _NC + lax.axis_index("c")
        base = wid * _TOK_W
        for j in range(_NCH):
            off = base + j * _CH
            pltpu.sync_copy(pos0_hbm.at[pl.ds(off, _CH)], idx_v.at[j])
            pltpu.sync_copy(pos1_hbm.at[pl.ds(off, _CH)], idx_v.at[_NCH + j])
            pltpu.sync_copy(hs_hbm.at[pl.ds(off, _CH)], rows_v)
            cp0 = pltpu.async_copy(rows_v, xs_hbm.at[idx_v.at[j]], sem0)
            cp1 = pltpu.async_copy(rows_v, xs_hbm.at[idx_v.at[_NCH + j]], sem1)
            cp0.wait()
            cp1.wait()

    @functools.partial(
        pl.kernel, mesh=mesh,
        out_type=jax.ShapeDtypeStruct((T, H), jnp.float32),
        scratch_types=[
            pltpu.VMEM((_CH, H), jnp.float32),
            pltpu.VMEM((_CH, H), jnp.float32),
            pltpu.VMEM((_CH, H), jnp.float32),
            pltpu.VMEM((_CH, 16), jnp.float32),
            pltpu.VMEM((_CH, 16), jnp.float32),
            pltpu.VMEM((2 * _NCH, _CH), jnp.int32),
            pltpu.SemaphoreType.DMA,
            pltpu.SemaphoreType.DMA,
        ],
    )
    def combine(y_hbm, pos0_hbm, pos1_hbm, w0_hbm, w1_hbm, out_hbm,
                rows0_v, rows1_v, outr_v, w0_v, w1_v, idx_v, sem0, sem1):
        wid = lax.axis_index("s") * _NC + lax.axis_index("c")
        base = wid * _TOK_W
        for j in range(_NCH):
            off = base + j * _CH
            pltpu.sync_copy(pos0_hbm.at[pl.ds(off, _CH)], idx_v.at[j])
            pltpu.sync_copy(pos1_hbm.at[pl.ds(off, _CH)], idx_v.at[_NCH + j])
            pltpu.sync_copy(w0_hbm.at[pl.ds(off, _CH)], w0_v)
            pltpu.sync_copy(w1_hbm.at[pl.ds(off, _CH)], w1_v)
            cp0 = pltpu.async_copy(y_hbm.at[idx_v.at[j]], rows0_v, sem0)
            cp1 = pltpu.async_copy(y_hbm.at[idx_v.at[_NCH + j]], rows1_v, sem1)
            cp0.wait()
            cp1.wait()
            for t in range(_CH):
                s0 = w0_v[t]
                s1 = w1_v[t]

                def vstep(v, _):
                    c = pl.multiple_of(v * 16, 16)
                    outr_v[t, pl.ds(c, 16)] = (
                        rows0_v[t, pl.ds(c, 16)] * s0
                        + rows1_v[t, pl.ds(c, 16)] * s1)
                    return 0

                lax.fori_loop(0, H // 16, vstep, 0)
            pltpu.sync_copy(outr_v, out_hbm.at[pl.ds(off, _CH)])

    return dispatch, combine


def _dispatch_sc(hs, p0, p1):
    return _make_sc_kernels()[0](hs, p0, p1)


def _combine_sc(y, p0, p1, w0, w1):
    return _make_sc_kernels()[1](y, p0, p1, w0, w1)


def _gemm_a_body(meta_ref, x_ref, wg_ref, wu_ref, h_ref):
    m = pl.program_id(0)

    @pl.when(m < meta_ref[NT])
    def _():
        xb = x_ref[...].astype(jnp.bfloat16)
        g = jnp.dot(xb, wg_ref[0].astype(jnp.bfloat16),
                    preferred_element_type=jnp.float32)
        u = jnp.dot(xb, wu_ref[0].astype(jnp.bfloat16),
                    preferred_element_type=jnp.float32)
        h_ref[...] = (u * (g * jax.nn.sigmoid(g))).astype(jnp.bfloat16)


def _gemm_b_body(meta_ref, h_ref, wd_ref, y_ref):
    m = pl.program_id(0)

    @pl.when(m < meta_ref[NT])
    def _():
        y_ref[...] = jnp.dot(h_ref[...], wd_ref[0].astype(jnp.bfloat16),
                             preferred_element_type=jnp.float32)


def _run_gemms(meta, x_sorted, gate_up_proj, down_proj):
    h = pl.pallas_call(
        _gemm_a_body,
        grid_spec=pltpu.PrefetchScalarGridSpec(
            num_scalar_prefetch=1,
            grid=(NT,),
            in_specs=[
                pl.BlockSpec((TM, H), lambda m, meta: (m, 0)),
                pl.BlockSpec((1, H, F), lambda m, meta: (meta[m], 0, 0)),
                pl.BlockSpec((1, H, F), lambda m, meta: (meta[m], 0, 1)),
            ],
            out_specs=pl.BlockSpec((TM, F), lambda m, meta: (m, 0)),
        ),
        out_shape=jax.ShapeDtypeStruct((NMAX, F), jnp.bfloat16),
    )(meta, x_sorted, gate_up_proj, gate_up_proj)

    y = pl.pallas_call(
        _gemm_b_body,
        grid_spec=pltpu.PrefetchScalarGridSpec(
            num_scalar_prefetch=1,
            grid=(NT,),
            in_specs=[
                pl.BlockSpec((TM, F), lambda m, meta: (m, 0)),
                pl.BlockSpec((1, F, H), lambda m, meta: (meta[m], 0, 0)),
            ],
            out_specs=pl.BlockSpec((TM, H), lambda m, meta: (m, 0)),
        ),
        out_shape=jax.ShapeDtypeStruct((NMAX, H), jnp.float32),
    )(meta, h, down_proj)
    return y


def _finish_body(w0_ref, w1_ref, y0_ref, y1_ref, out_ref):
    out_ref[...] = w0_ref[...] * y0_ref[...] + w1_ref[...] * y1_ref[...]


def _run_finish(w0, w1, y0, y1):
    bt = T // 4
    return pl.pallas_call(
        _finish_body,
        grid=(4,),
        in_specs=[
            pl.BlockSpec((bt, 1), lambda i: (i, 0)),
            pl.BlockSpec((bt, 1), lambda i: (i, 0)),
            pl.BlockSpec((bt, H), lambda i: (i, 0)),
            pl.BlockSpec((bt, H), lambda i: (i, 0)),
        ],
        out_specs=pl.BlockSpec((bt, H), lambda i: (i, 0)),
        out_shape=jax.ShapeDtypeStruct((T, H), jnp.float32),
    )(w0, w1, y0, y1)


def kernel(hidden_states, gate_weight, gate_up_proj, down_proj):
    B, S, _ = hidden_states.shape
    hs = hidden_states.reshape(T, H)
    gwT = gate_weight.T

    logits, w0, w1, pos0, pos1, sizes, xbf = _run_router(hs, gwT)

    sizes_v = sizes[0]
    psz = ((sizes_v + TM - 1) // TM) * TM
    ends = jnp.cumsum(psz)
    n_act = (ends[-1] // TM).astype(jnp.int32)
    it = jnp.arange(NT, dtype=jnp.int32) * TM
    raw = jnp.sum((it[:, None] >= ends[None, :]).astype(jnp.int32), axis=1)
    te_raw = jnp.clip(raw, 0, E - 1)
    last_e = te_raw[jnp.maximum(n_act - 1, 0)]
    active = jnp.arange(NT, dtype=jnp.int32) < n_act
    te = jnp.where(active, te_raw, last_e).astype(jnp.int32)
    meta = jnp.concatenate([te, n_act[None]])

    p0 = pos0.reshape(T)
    p1 = pos1.reshape(T)

    x_sorted = _dispatch_sc(hs, p0, p1)
    y = _run_gemms(meta, x_sorted, gate_up_proj, down_proj)
    out = _combine_sc(y, p0, p1, w0, w1)

    return out.reshape(B, S, H), logits


# speedup vs baseline: 1.4430x; 1.0768x over previous
import functools

import jax
import jax.numpy as jnp
from jax import lax
from jax.experimental import pallas as pl
from jax.experimental.pallas import tpu as pltpu
from jax.experimental.pallas import tpu_sc as plsc

E = 8
TOP_K = 2
H = 2048
F = 1024
T = 2048
TM = 256
NMAX = T * TOP_K + E * TM
NT = NMAX // TM

_NC = 2
_NS = 16
_NW = _NC * _NS
_TOK_W = T // _NW
_CH = 16
_NCH = _TOK_W // _CH


def _router_body(x_ref, gwT_ref, logits_ref, w0_ref, w1_ref,
                 pos0_ref, pos1_ref, sizes_ref, xbf_ref):
    x = x_ref[...]
    xb = x.astype(jnp.bfloat16)
    xbf_ref[...] = xb
    logits = jnp.dot(xb, gwT_ref[...].astype(jnp.bfloat16),
                     preferred_element_type=jnp.float32)
    logits_ref[...] = logits
    p = jax.nn.softmax(logits, axis=-1)
    lane = lax.broadcasted_iota(jnp.int32, p.shape, 1)
    p1 = jnp.max(p, axis=-1, keepdims=True)
    i1 = jnp.min(jnp.where(p == p1, lane, E), axis=-1, keepdims=True)
    m1 = lane == i1
    p_no1 = jnp.where(m1, -jnp.inf, p)
    p2 = jnp.max(p_no1, axis=-1, keepdims=True)
    i2 = jnp.min(jnp.where(p_no1 == p2, lane, E), axis=-1, keepdims=True)
    m2 = lane == i2
    denom = p1 + p2
    w0_ref[...] = p1 / denom
    w1_ref[...] = p2 / denom

    C = jnp.where(m1 | m2, 1.0, 0.0)
    r128 = lax.broadcasted_iota(jnp.int32, (TM, TM), 0)
    c128 = lax.broadcasted_iota(jnp.int32, (TM, TM), 1)
    L128 = jnp.where(r128 > c128, 1.0, 0.0).astype(jnp.bfloat16)
    excl_rows = []
    acc = jnp.zeros((1, E), jnp.float32)
    for b in range(T // TM):
        C_sl = C[b * TM:(b + 1) * TM]
        intra = jnp.dot(L128, C_sl.astype(jnp.bfloat16),
                        preferred_element_type=jnp.float32)
        excl_rows.append(intra + acc)
        acc = acc + jnp.sum(C_sl, axis=0, keepdims=True)
    excl = jnp.concatenate(excl_rows, axis=0)
    sizes = acc

    q = jnp.ceil(sizes / TM)
    r8 = lax.broadcasted_iota(jnp.int32, (E, E), 0)
    c8 = lax.broadcasted_iota(jnp.int32, (E, E), 1)
    U8 = jnp.where(r8 < c8, 1.0, 0.0).astype(jnp.bfloat16)
    starts = TM * jnp.dot(q.astype(jnp.bfloat16), U8,
                          preferred_element_type=jnp.float32)

    starts_b = jnp.broadcast_to(starts, (T, E))
    sel1 = jnp.sum(jnp.where(lane == i1, starts_b + excl, 0.0),
                   axis=-1, keepdims=True)
    sel2 = jnp.sum(jnp.where(lane == i2, starts_b + excl, 0.0),
                   axis=-1, keepdims=True)
    pos0_ref[...] = sel1.astype(jnp.int32)
    pos1_ref[...] = sel2.astype(jnp.int32)
    sizes_ref[...] = sizes.astype(jnp.int32)


def _run_router(hs, gwT):
    return pl.pallas_call(
        _router_body,
        in_specs=[
            pl.BlockSpec((T, H), lambda: (0, 0)),
            pl.BlockSpec((H, E), lambda: (0, 0)),
        ],
        out_specs=[
            pl.BlockSpec((T, E), lambda: (0, 0)),
            pl.BlockSpec((T, 1), lambda: (0, 0)),
            pl.BlockSpec((T, 1), lambda: (0, 0)),
            pl.BlockSpec((T, 1), lambda: (0, 0)),
            pl.BlockSpec((T, 1), lambda: (0, 0)),
            pl.BlockSpec((1, E), lambda: (0, 0)),
            pl.BlockSpec((T, H), lambda: (0, 0)),
        ],
        out_shape=[
            jax.ShapeDtypeStruct((T, E), jnp.float32),
            jax.ShapeDtypeStruct((T, 1), jnp.float32),
            jax.ShapeDtypeStruct((T, 1), jnp.float32),
            jax.ShapeDtypeStruct((T, 1), jnp.int32),
            jax.ShapeDtypeStruct((T, 1), jnp.int32),
            jax.ShapeDtypeStruct((1, E), jnp.int32),
            jax.ShapeDtypeStruct((T, H), jnp.bfloat16),
        ],
    )(hs, gwT)


@functools.cache
def _make_sc_kernels():
    mesh = plsc.VectorSubcoreMesh(core_axis_name="c", subcore_axis_name="s")

    @functools.partial(
        pl.kernel, mesh=mesh,
        out_type=jax.ShapeDtypeStruct((NMAX, H), jnp.float32),
        scratch_types=[
            pltpu.VMEM((_CH, H), jnp.float32),
            pltpu.VMEM((2 * _NCH, _CH), jnp.int32),
            pltpu.SemaphoreType.DMA,
            pltpu.SemaphoreType.DMA,
        ],
    )
    def dispatch(hs_hbm, pos0_hbm, pos1_hbm, xs_hbm, rows_v, idx_v, sem0, sem1):
        wid = lax.axis_index("s") * _NC + lax.axis_index("c")
        base = wid * _TOK_W
        for j in range(_NCH):
            off = base + j * _CH
            pltpu.sync_copy(pos0_hbm.at[pl.ds(off, _CH)], idx_v.at[j])
            pltpu.sync_copy(pos1_hbm.at[pl.ds(off, _CH)], idx_v.at[_NCH + j])
            pltpu.sync_copy(hs_hbm.at[pl.ds(off, _CH)], rows_v)
            cp0 = pltpu.async_copy(rows_v, xs_hbm.at[idx_v.at[j]], sem0)
            cp1 = pltpu.async_copy(rows_v, xs_hbm.at[idx_v.at[_NCH + j]], sem1)
            cp0.wait()
            cp1.wait()

    @functools.partial(
        pl.kernel, mesh=mesh,
        out_type=[
            jax.ShapeDtypeStruct((T, H), jnp.float32),
            jax.ShapeDtypeStruct((T, H), jnp.float32),
        ],
        scratch_types=[
            pltpu.VMEM((_CH, H), jnp.float32),
            pltpu.VMEM((_CH, H), jnp.float32),
            pltpu.VMEM((2 * _NCH, _CH), jnp.int32),
            pltpu.SemaphoreType.DMA,
            pltpu.SemaphoreType.DMA,
        ],
    )
    def combine(y_hbm, pos0_hbm, pos1_hbm, y0_hbm, y1_hbm,
                rows0_v, rows1_v, idx_v, sem0, sem1):
        wid = lax.axis_index("s") * _NC + lax.axis_index("c")
        base = wid * _TOK_W
        for j in range(_NCH):
            off = base + j * _CH
            pltpu.sync_copy(pos0_hbm.at[pl.ds(off, _CH)], idx_v.at[j])
            pltpu.sync_copy(pos1_hbm.at[pl.ds(off, _CH)], idx_v.at[_NCH + j])
            cp0 = pltpu.async_copy(y_hbm.at[idx_v.at[j]], rows0_v, sem0)
            cp1 = pltpu.async_copy(y_hbm.at[idx_v.at[_NCH + j]], rows1_v, sem1)
            cp0.wait()
            cp1.wait()
            pltpu.sync_copy(rows0_v, y0_hbm.at[pl.ds(off, _CH)])
            pltpu.sync_copy(rows1_v, y1_hbm.at[pl.ds(off, _CH)])

    return dispatch, combine


def _dispatch_sc(hs, p0, p1):
    return _make_sc_kernels()[0](hs, p0, p1)


def _combine_sc(y, p0, p1):
    return _make_sc_kernels()[1](y, p0, p1)


def _gemm_a_body(meta_ref, x_ref, wg_ref, wu_ref, h_ref):
    m = pl.program_id(0)

    @pl.when(m < meta_ref[NT])
    def _():
        xb = x_ref[...].astype(jnp.bfloat16)
        g = jnp.dot(xb, wg_ref[0].astype(jnp.bfloat16),
                    preferred_element_type=jnp.float32)
        u = jnp.dot(xb, wu_ref[0].astype(jnp.bfloat16),
                    preferred_element_type=jnp.float32)
        h_ref[...] = (u * (g * jax.nn.sigmoid(g))).astype(jnp.bfloat16)


def _gemm_b_body(meta_ref, h_ref, wd_ref, y_ref):
    m = pl.program_id(0)

    @pl.when(m < meta_ref[NT])
    def _():
        y_ref[...] = jnp.dot(h_ref[...], wd_ref[0].astype(jnp.bfloat16),
                             preferred_element_type=jnp.float32)


def _run_gemms(meta, x_sorted, gate_up_proj, down_proj):
    h = pl.pallas_call(
        _gemm_a_body,
        grid_spec=pltpu.PrefetchScalarGridSpec(
            num_scalar_prefetch=1,
            grid=(NT,),
            in_specs=[
                pl.BlockSpec((TM, H), lambda m, meta: (m, 0)),
                pl.BlockSpec((1, H, F), lambda m, meta: (meta[m], 0, 0)),
                pl.BlockSpec((1, H, F), lambda m, meta: (meta[m], 0, 1)),
            ],
            out_specs=pl.BlockSpec((TM, F), lambda m, meta: (m, 0)),
        ),
        out_shape=jax.ShapeDtypeStruct((NMAX, F), jnp.bfloat16),
    )(meta, x_sorted, gate_up_proj, gate_up_proj)

    y = pl.pallas_call(
        _gemm_b_body,
        grid_spec=pltpu.PrefetchScalarGridSpec(
            num_scalar_prefetch=1,
            grid=(NT,),
            in_specs=[
                pl.BlockSpec((TM, F), lambda m, meta: (m, 0)),
                pl.BlockSpec((1, F, H), lambda m, meta: (meta[m], 0, 0)),
            ],
            out_specs=pl.BlockSpec((TM, H), lambda m, meta: (m, 0)),
        ),
        out_shape=jax.ShapeDtypeStruct((NMAX, H), jnp.float32),
    )(meta, h, down_proj)
    return y


def _finish_body(w0_ref, w1_ref, y0_ref, y1_ref, out_ref):
    out_ref[...] = w0_ref[...] * y0_ref[...] + w1_ref[...] * y1_ref[...]


def _run_finish(w0, w1, y0, y1):
    bt = T // 4
    return pl.pallas_call(
        _finish_body,
        grid=(4,),
        in_specs=[
            pl.BlockSpec((bt, 1), lambda i: (i, 0)),
            pl.BlockSpec((bt, 1), lambda i: (i, 0)),
            pl.BlockSpec((bt, H), lambda i: (i, 0)),
            pl.BlockSpec((bt, H), lambda i: (i, 0)),
        ],
        out_specs=pl.BlockSpec((bt, H), lambda i: (i, 0)),
        out_shape=jax.ShapeDtypeStruct((T, H), jnp.float32),
    )(w0, w1, y0, y1)


def kernel(hidden_states, gate_weight, gate_up_proj, down_proj):
    B, S, _ = hidden_states.shape
    hs = hidden_states.reshape(T, H)
    gwT = gate_weight.T

    logits, w0, w1, pos0, pos1, sizes, xbf = _run_router(hs, gwT)

    sizes_v = sizes[0]
    psz = ((sizes_v + TM - 1) // TM) * TM
    ends = jnp.cumsum(psz)
    n_act = (ends[-1] // TM).astype(jnp.int32)
    it = jnp.arange(NT, dtype=jnp.int32) * TM
    raw = jnp.sum((it[:, None] >= ends[None, :]).astype(jnp.int32), axis=1)
    te_raw = jnp.clip(raw, 0, E - 1)
    last_e = te_raw[jnp.maximum(n_act - 1, 0)]
    active = jnp.arange(NT, dtype=jnp.int32) < n_act
    te = jnp.where(active, te_raw, last_e).astype(jnp.int32)
    meta = jnp.concatenate([te, n_act[None]])

    p0 = pos0.reshape(T)
    p1 = pos1.reshape(T)

    x_sorted = _dispatch_sc(hs, p0, p1)
    y = _run_gemms(meta, x_sorted, gate_up_proj, down_proj)
    y0, y1 = _combine_sc(y, p0, p1)
    out = _run_finish(w0, w1, y0, y1)

    return out.reshape(B, S, H), logits


# speedup vs baseline: 1.4469x; 1.0027x over previous
import functools

import jax
import jax.numpy as jnp
from jax import lax
from jax.experimental import pallas as pl
from jax.experimental.pallas import tpu as pltpu
from jax.experimental.pallas import tpu_sc as plsc

E = 8
TOP_K = 2
H = 2048
F = 1024
T = 2048
TM = 256
NMAX = T * TOP_K + E * TM
NT = NMAX // TM

_NC = 2
_NS = 16
_NW = _NC * _NS
_TOK_W = T // _NW
_CH = 16
_NCH = _TOK_W // _CH
_CHC = 8
_NCHC = _TOK_W // _CHC


def _router_body(x_ref, gwT_ref, logits_ref, w0_ref, w1_ref,
                 pos0_ref, pos1_ref, sizes_ref, xbf_ref):
    x = x_ref[...]
    xb = x.astype(jnp.bfloat16)
    xbf_ref[...] = xb
    logits = jnp.dot(xb, gwT_ref[...].astype(jnp.bfloat16),
                     preferred_element_type=jnp.float32)
    logits_ref[...] = logits
    p = jax.nn.softmax(logits, axis=-1)
    lane = lax.broadcasted_iota(jnp.int32, p.shape, 1)
    p1 = jnp.max(p, axis=-1, keepdims=True)
    i1 = jnp.min(jnp.where(p == p1, lane, E), axis=-1, keepdims=True)
    m1 = lane == i1
    p_no1 = jnp.where(m1, -jnp.inf, p)
    p2 = jnp.max(p_no1, axis=-1, keepdims=True)
    i2 = jnp.min(jnp.where(p_no1 == p2, lane, E), axis=-1, keepdims=True)
    m2 = lane == i2
    denom = p1 + p2
    w0_ref[...] = p1 / denom
    w1_ref[...] = p2 / denom

    C = jnp.where(m1 | m2, 1.0, 0.0)
    r128 = lax.broadcasted_iota(jnp.int32, (TM, TM), 0)
    c128 = lax.broadcasted_iota(jnp.int32, (TM, TM), 1)
    L128 = jnp.where(r128 > c128, 1.0, 0.0).astype(jnp.bfloat16)
    excl_rows = []
    acc = jnp.zeros((1, E), jnp.float32)
    for b in range(T // TM):
        C_sl = C[b * TM:(b + 1) * TM]
        intra = jnp.dot(L128, C_sl.astype(jnp.bfloat16),
                        preferred_element_type=jnp.float32)
        excl_rows.append(intra + acc)
        acc = acc + jnp.sum(C_sl, axis=0, keepdims=True)
    excl = jnp.concatenate(excl_rows, axis=0)
    sizes = acc

    q = jnp.ceil(sizes / TM)
    r8 = lax.broadcasted_iota(jnp.int32, (E, E), 0)
    c8 = lax.broadcasted_iota(jnp.int32, (E, E), 1)
    U8 = jnp.where(r8 < c8, 1.0, 0.0).astype(jnp.bfloat16)
    starts = TM * jnp.dot(q.astype(jnp.bfloat16), U8,
                          preferred_element_type=jnp.float32)

    starts_b = jnp.broadcast_to(starts, (T, E))
    sel1 = jnp.sum(jnp.where(lane == i1, starts_b + excl, 0.0),
                   axis=-1, keepdims=True)
    sel2 = jnp.sum(jnp.where(lane == i2, starts_b + excl, 0.0),
                   axis=-1, keepdims=True)
    pos0_ref[...] = sel1.astype(jnp.int32)
    pos1_ref[...] = sel2.astype(jnp.int32)
    sizes_ref[...] = sizes.astype(jnp.int32)


def _run_router(hs, gwT):
    return pl.pallas_call(
        _router_body,
        in_specs=[
            pl.BlockSpec((T, H), lambda: (0, 0)),
            pl.BlockSpec((H, E), lambda: (0, 0)),
        ],
        out_specs=[
            pl.BlockSpec((T, E), lambda: (0, 0)),
            pl.BlockSpec((T, 1), lambda: (0, 0)),
            pl.BlockSpec((T, 1), lambda: (0, 0)),
            pl.BlockSpec((T, 1), lambda: (0, 0)),
            pl.BlockSpec((T, 1), lambda: (0, 0)),
            pl.BlockSpec((1, E), lambda: (0, 0)),
            pl.BlockSpec((T, H), lambda: (0, 0)),
        ],
        out_shape=[
            jax.ShapeDtypeStruct((T, E), jnp.float32),
            jax.ShapeDtypeStruct((T, 1), jnp.float32),
            jax.ShapeDtypeStruct((T, 1), jnp.float32),
            jax.ShapeDtypeStruct((T, 1), jnp.int32),
            jax.ShapeDtypeStruct((T, 1), jnp.int32),
            jax.ShapeDtypeStruct((1, E), jnp.int32),
            jax.ShapeDtypeStruct((T, H), jnp.bfloat16),
        ],
    )(hs, gwT)


@functools.cache
def _make_sc_kernels():
    mesh = plsc.VectorSubcoreMesh(core_axis_name="c", subcore_axis_name="s")

    @functools.partial(
        pl.kernel, mesh=mesh,
        out_type=jax.ShapeDtypeStruct((NMAX, H), jnp.float32),
        scratch_types=[
            pltpu.VMEM((2, _CH, H), jnp.float32),
            pltpu.VMEM((2 * _NCH, _CH), jnp.int32),
            pltpu.SemaphoreType.DMA,
            pltpu.SemaphoreType.DMA,
        ],
    )
    def dispatch(hs_hbm, pos0_hbm, pos1_hbm, xs_hbm, rows_v, idx_v, sem0, sem1):
        wid = lax.axis_index("s") * _NC + lax.axis_index("c")
        base = wid * _TOK_W
        for j in range(_NCH):
            off = base + j * _CH
            pltpu.sync_copy(pos0_hbm.at[pl.ds(off, _CH)], idx_v.at[j])
            pltpu.sync_copy(pos1_hbm.at[pl.ds(off, _CH)], idx_v.at[_NCH + j])
        pending = {}
        for j in range(_NCH):
            off = base + j * _CH
            b = j % 2
            if j >= 2:
                pending[j - 2][0].wait()
                pending[j - 2][1].wait()
            pltpu.sync_copy(hs_hbm.at[pl.ds(off, _CH)], rows_v.at[b])
            cp0 = pltpu.async_copy(rows_v.at[b], xs_hbm.at[idx_v.at[j]], sem0)
            cp1 = pltpu.async_copy(rows_v.at[b], xs_hbm.at[idx_v.at[_NCH + j]],
                                   sem1)
            pending[j] = (cp0, cp1)
        for j in range(_NCH - 2, _NCH):
            pending[j][0].wait()
            pending[j][1].wait()

    @functools.partial(
        pl.kernel, mesh=mesh,
        out_type=[
            jax.ShapeDtypeStruct((T, H), jnp.float32),
            jax.ShapeDtypeStruct((T, H), jnp.float32),
        ],
        scratch_types=[
            pltpu.VMEM((2, _CHC, H), jnp.float32),
            pltpu.VMEM((2, _CHC, H), jnp.float32),
            pltpu.VMEM((2 * _NCHC, _CHC), jnp.int32),
            pltpu.SemaphoreType.DMA,
            pltpu.SemaphoreType.DMA,
        ],
    )
    def combine(y_hbm, pos0_hbm, pos1_hbm, y0_hbm, y1_hbm,
                rows0_v, rows1_v, idx_v, sem0, sem1):
        wid = lax.axis_index("s") * _NC + lax.axis_index("c")
        base = wid * _TOK_W
        for j in range(_NCHC):
            off = base + j * _CHC
            pltpu.sync_copy(pos0_hbm.at[pl.ds(off, _CHC)], idx_v.at[j])
            pltpu.sync_copy(pos1_hbm.at[pl.ds(off, _CHC)], idx_v.at[_NCHC + j])
        pending = {}
        for j in range(_NCHC):
            b = j % 2
            if j >= 2:
                pending[j - 2][0].wait()
                pending[j - 2][1].wait()
                off2 = base + (j - 2) * _CHC
                pltpu.sync_copy(rows0_v.at[j % 2], y0_hbm.at[pl.ds(off2, _CHC)])
                pltpu.sync_copy(rows1_v.at[j % 2], y1_hbm.at[pl.ds(off2, _CHC)])
            cp0 = pltpu.async_copy(y_hbm.at[idx_v.at[j]], rows0_v.at[b], sem0)
            cp1 = pltpu.async_copy(y_hbm.at[idx_v.at[_NCHC + j]], rows1_v.at[b],
                                   sem1)
            pending[j] = (cp0, cp1)
        for j in range(_NCHC - 2, _NCHC):
            pending[j][0].wait()
            pending[j][1].wait()
            off2 = base + j * _CHC
            pltpu.sync_copy(rows0_v.at[j % 2], y0_hbm.at[pl.ds(off2, _CHC)])
            pltpu.sync_copy(rows1_v.at[j % 2], y1_hbm.at[pl.ds(off2, _CHC)])

    return dispatch, combine


def _dispatch_sc(hs, p0, p1):
    return _make_sc_kernels()[0](hs, p0, p1)


def _combine_sc(y, p0, p1):
    return _make_sc_kernels()[1](y, p0, p1)


def _gemm_a_body(meta_ref, x_ref, wg_ref, wu_ref, h_ref):
    m = pl.program_id(0)

    @pl.when(m < meta_ref[NT])
    def _():
        xb = x_ref[...].astype(jnp.bfloat16)
        g = jnp.dot(xb, wg_ref[0].astype(jnp.bfloat16),
                    preferred_element_type=jnp.float32)
        u = jnp.dot(xb, wu_ref[0].astype(jnp.bfloat16),
                    preferred_element_type=jnp.float32)
        h_ref[...] = (u * (g * jax.nn.sigmoid(g))).astype(jnp.bfloat16)


def _gemm_b_body(meta_ref, h_ref, wd_ref, y_ref):
    m = pl.program_id(0)

    @pl.when(m < meta_ref[NT])
    def _():
        y_ref[...] = jnp.dot(h_ref[...], wd_ref[0].astype(jnp.bfloat16),
                             preferred_element_type=jnp.float32)


def _run_gemms(meta, x_sorted, gate_up_proj, down_proj):
    h = pl.pallas_call(
        _gemm_a_body,
        grid_spec=pltpu.PrefetchScalarGridSpec(
            num_scalar_prefetch=1,
            grid=(NT,),
            in_specs=[
                pl.BlockSpec((TM, H), lambda m, meta: (m, 0)),
                pl.BlockSpec((1, H, F), lambda m, meta: (meta[m], 0, 0)),
                pl.BlockSpec((1, H, F), lambda m, meta: (meta[m], 0, 1)),
            ],
            out_specs=pl.BlockSpec((TM, F), lambda m, meta: (m, 0)),
        ),
        out_shape=jax.ShapeDtypeStruct((NMAX, F), jnp.bfloat16),
    )(meta, x_sorted, gate_up_proj, gate_up_proj)

    y = pl.pallas_call(
        _gemm_b_body,
        grid_spec=pltpu.PrefetchScalarGridSpec(
            num_scalar_prefetch=1,
            grid=(NT,),
            in_specs=[
                pl.BlockSpec((TM, F), lambda m, meta: (m, 0)),
                pl.BlockSpec((1, F, H), lambda m, meta: (meta[m], 0, 0)),
            ],
            out_specs=pl.BlockSpec((TM, H), lambda m, meta: (m, 0)),
        ),
        out_shape=jax.ShapeDtypeStruct((NMAX, H), jnp.float32),
    )(meta, h, down_proj)
    return y


def _finish_body(w0_ref, w1_ref, y0_ref, y1_ref, out_ref):
    out_ref[...] = w0_ref[...] * y0_ref[...] + w1_ref[...] * y1_ref[...]


def _run_finish(w0, w1, y0, y1):
    bt = T // 4
    return pl.pallas_call(
        _finish_body,
        grid=(4,),
        in_specs=[
            pl.BlockSpec((bt, 1), lambda i: (i, 0)),
            pl.BlockSpec((bt, 1), lambda i: (i, 0)),
            pl.BlockSpec((bt, H), lambda i: (i, 0)),
            pl.BlockSpec((bt, H), lambda i: (i, 0)),
        ],
        out_specs=pl.BlockSpec((bt, H), lambda i: (i, 0)),
        out_shape=jax.ShapeDtypeStruct((T, H), jnp.float32),
    )(w0, w1, y0, y1)


def kernel(hidden_states, gate_weight, gate_up_proj, down_proj):
    B, S, _ = hidden_states.shape
    hs = hidden_states.reshape(T, H)
    gwT = gate_weight.T

    logits, w0, w1, pos0, pos1, sizes, xbf = _run_router(hs, gwT)

    sizes_v = sizes[0]
    psz = ((sizes_v + TM - 1) // TM) * TM
    ends = jnp.cumsum(psz)
    n_act = (ends[-1] // TM).astype(jnp.int32)
    it = jnp.arange(NT, dtype=jnp.int32) * TM
    raw = jnp.sum((it[:, None] >= ends[None, :]).astype(jnp.int32), axis=1)
    te_raw = jnp.clip(raw, 0, E - 1)
    last_e = te_raw[jnp.maximum(n_act - 1, 0)]
    active = jnp.arange(NT, dtype=jnp.int32) < n_act
    te = jnp.where(active, te_raw, last_e).astype(jnp.int32)
    meta = jnp.concatenate([te, n_act[None]])

    p0 = pos0.reshape(T)
    p1 = pos1.reshape(T)

    x_sorted = _dispatch_sc(hs, p0, p1)
    y = _run_gemms(meta, x_sorted, gate_up_proj, down_proj)
    y0, y1 = _combine_sc(y, p0, p1)
    out = _run_finish(w0, w1, y0, y1)

    return out.reshape(B, S, H), logits
